# baseline (speedup 1.0000x reference)
import jax
import jax.numpy as jnp
from jax.experimental import pallas as pl
from jax.experimental.pallas import tpu as pltpu

_CHUNK = 2000
_NCHUNK = 5


def _fused_kernel(x_hbm, wcat_ref, bcat_ref, wlin_ref, blin_ref,
                  out_ref, buf, sems):
    copies = [
        pltpu.make_async_copy(
            x_hbm.at[pl.ds(ci * _CHUNK, _CHUNK), :], buf.at[ci], sems.at[ci])
        for ci in range(_NCHUNK)
    ]
    for c in copies:
        c.start()
    wcat = wcat_ref[...]
    bcat = bcat_ref[...]
    wlin = wlin_ref[...]
    blin = blin_ref[...]
    hid = wlin.shape[0]
    for ci in range(_NCHUNK):
        copies[ci].wait()
        x = buf[ci].astype(jnp.bfloat16)
        lg = jax.lax.dot_general(wcat, x, (((0,), (1,)), ((), ())),
                                 preferred_element_type=jnp.float32) + bcat
        tau = jnp.tanh(lg)
        s1 = 1.0 + tau[:hid]
        t = tau[hid:]
        h = jax.nn.relu(s1 * t)
        o = jnp.sum(h * wlin, axis=0, keepdims=True)
        out_ref[ci] = o + blin


def kernel(x, edge_index, edge_weight, W_xz, b_xz, W_hz, b_hz, W_xr, b_xr,
           W_hr, b_hr, W_xh, b_xh, W_hh, b_hh, W_lin, b_lin):
    n, f_in = x.shape
    hid = W_xz.shape[1]
    wcat = jnp.concatenate([-0.5 * W_xz, W_xh], axis=1).astype(jnp.bfloat16)
    bcat = jnp.concatenate([-0.5 * (b_xz + b_hz), b_xh + b_hh]).reshape(
        2 * hid, 1)
    wlin = (0.5 * W_lin).reshape(hid, 1)
    blin = b_lin.reshape(1, 1)

    vm = pl.BlockSpec(memory_space=pltpu.MemorySpace.VMEM)
    out_row = pl.pallas_call(
        _fused_kernel,
        in_specs=[
            pl.BlockSpec(memory_space=pltpu.MemorySpace.HBM),
            vm, vm, vm, vm,
        ],
        out_specs=vm,
        out_shape=jax.ShapeDtypeStruct((_NCHUNK, 1, _CHUNK), x.dtype),
        scratch_shapes=[
            pltpu.MemorySpace.VMEM((_NCHUNK, _CHUNK, f_in), jnp.float32),
            pltpu.SemaphoreType.DMA((_NCHUNK,)),
        ],
    )(x, wcat, bcat, wlin, blin)
    return out_row.reshape(n, 1)


# speedup vs baseline: 1.1295x; 1.1295x over previous
import jax
import jax.numpy as jnp
from jax.experimental import pallas as pl
from jax.experimental.pallas import tpu as pltpu

_CHUNK = 5000
_NCHUNK = 2


def _fused_kernel(x_hbm, pk_ref, out_ref, buf, sems):
    copies = [
        pltpu.make_async_copy(
            x_hbm.at[pl.ds(ci * _CHUNK, _CHUNK), :], buf.at[ci], sems.at[ci])
        for ci in range(_NCHUNK)
    ]
    for c in copies:
        c.start()
    wcat = pk_ref[0:128, :].astype(jnp.bfloat16)
    bcat = pk_ref[128:192, 0:1]
    wlin = pk_ref[192:224, 0:1]
    blin = pk_ref[224:225, 0:1]
    hid = wlin.shape[0]
    for ci in range(_NCHUNK):
        copies[ci].wait()
        x = buf[ci].astype(jnp.bfloat16)
        lg = jax.lax.dot_general(wcat, x, (((0,), (1,)), ((), ())),
                                 preferred_element_type=jnp.float32) + bcat
        tau = jnp.tanh(lg)
        s1 = 1.0 + tau[:hid]
        t = tau[hid:]
        h = jax.nn.relu(s1 * t)
        o = jnp.sum(h * wlin, axis=0, keepdims=True)
        out_ref[ci] = o + blin


def kernel(x, edge_index, edge_weight, W_xz, b_xz, W_hz, b_hz, W_xr, b_xr,
           W_hr, b_hr, W_xh, b_xh, W_hh, b_hh, W_lin, b_lin):
    n, f_in = x.shape
    hid = W_xz.shape[1]
    wcat = jnp.concatenate([-0.5 * W_xz, W_xh], axis=1)
    tail = jnp.concatenate([
        -0.5 * (b_xz + b_hz), b_xh + b_hh,
        0.5 * W_lin[:, 0],
        b_lin,
    ]).reshape(2 * hid + hid + 1, 1)
    pk = jnp.concatenate(
        [wcat, jnp.pad(tail, ((0, 0), (0, 2 * hid - 1)))], axis=0)

    vm = pl.BlockSpec(memory_space=pltpu.MemorySpace.VMEM)
    out_row = pl.pallas_call(
        _fused_kernel,
        in_specs=[
            pl.BlockSpec(memory_space=pltpu.MemorySpace.HBM),
            vm,
        ],
        out_specs=vm,
        out_shape=jax.ShapeDtypeStruct((_NCHUNK, 1, _CHUNK), x.dtype),
        scratch_shapes=[
            pltpu.MemorySpace.VMEM((_NCHUNK, _CHUNK, f_in), jnp.float32),
            pltpu.SemaphoreType.DMA((_NCHUNK,)),
        ],
    )(x, pk)
    return out_row.reshape(n, 1)
